# Initial kernel scaffold; baseline (speedup 1.0000x reference)
import jax
import jax.numpy as jnp
from jax.experimental import pallas as pl


def kernel(logits):
    raise NotImplementedError("write your pallas kernel here")


# speedup vs baseline: 31.3168x; 31.3168x over previous
import jax
import jax.numpy as jnp
from jax.experimental import pallas as pl
from jax.experimental.pallas import tpu as pltpu

_TOP_P = 0.9
_FILTER_VALUE = -1e9
_ROWS_PER_BLOCK = 8
_N_ITERS = 26


def _nucleus_block(x_ref, o_ref, e_ref):
    x = x_ref[...]
    m = jnp.max(x, axis=-1, keepdims=True)
    e = jnp.exp(x - m)
    e_ref[...] = e
    z = jnp.sum(e, axis=-1, keepdims=True)
    target = _TOP_P * z

    def body(_, carry):
        lo, hi = carry
        mid = 0.5 * (lo + hi)
        tau = jnp.exp(mid)
        ee = e_ref[...]
        f = jnp.sum(jnp.where(ee > tau, ee, 0.0), axis=-1, keepdims=True)
        gt = f > target
        return jnp.where(gt, mid, lo), jnp.where(gt, hi, mid)

    lo0 = jnp.full_like(z, -25.0)
    hi0 = jnp.zeros_like(z)
    lo, _ = jax.lax.fori_loop(0, _N_ITERS, body, (lo0, hi0))

    tau_lo = jnp.exp(lo)
    ee = e_ref[...]
    keep = ee > tau_lo
    zk = jnp.sum(jnp.where(keep, ee, 0.0), axis=-1, keepdims=True)
    lzk = jnp.log(zk)
    y = x_ref[...] - m
    o_ref[...] = jnp.where(keep, y - lzk, (_FILTER_VALUE - m) - lzk)


def kernel(logits):
    n_rows, vocab = logits.shape
    vp = ((vocab + 127) // 128) * 128
    xp = jnp.pad(logits, ((0, 0), (0, vp - vocab)), constant_values=-1e30)
    grid = (n_rows // _ROWS_PER_BLOCK,)
    out = pl.pallas_call(
        _nucleus_block,
        grid=grid,
        in_specs=[pl.BlockSpec((_ROWS_PER_BLOCK, vp), lambda i: (i, 0))],
        out_specs=pl.BlockSpec((_ROWS_PER_BLOCK, vp), lambda i: (i, 0)),
        out_shape=jax.ShapeDtypeStruct((n_rows, vp), jnp.float32),
        scratch_shapes=[pltpu.VMEM((_ROWS_PER_BLOCK, vp), jnp.float32)],
    )(xp)
    return out[:, :vocab]


# speedup vs baseline: 37.0145x; 1.1819x over previous
import jax
import jax.numpy as jnp
from jax.experimental import pallas as pl
from jax.experimental.pallas import tpu as pltpu

_TOP_P = 0.9
_FILTER_VALUE = -1e9
_ROWS_PER_BLOCK = 8
_N_ITERS = 20


def _nucleus_block(x_ref, o_ref, e_ref):
    x = x_ref[...]
    m = jnp.max(x, axis=-1, keepdims=True)
    e = jnp.exp(x - m)
    e_ref[...] = e
    z = jnp.sum(e, axis=-1, keepdims=True)
    target = _TOP_P * z

    def body(_, carry):
        lo, hi = carry
        mid = 0.5 * (lo + hi)
        tau = jnp.exp(mid)
        ee = e_ref[...]
        f = jnp.sum(jnp.where(ee > tau, ee, 0.0), axis=-1, keepdims=True)
        gt = f > target
        return jnp.where(gt, mid, lo), jnp.where(gt, hi, mid)

    lo0 = jnp.full_like(z, -25.0)
    hi0 = jnp.zeros_like(z)
    lo, _ = jax.lax.fori_loop(0, _N_ITERS, body, (lo0, hi0))

    tau_lo = jnp.exp(lo)
    ee = e_ref[...]
    keep = ee > tau_lo
    zk = jnp.sum(jnp.where(keep, ee, 0.0), axis=-1, keepdims=True)
    lzk = jnp.log(zk)
    y = x_ref[...] - m
    o_ref[...] = jnp.where(keep, y - lzk, (_FILTER_VALUE - m) - lzk)


def kernel(logits):
    n_rows, vocab = logits.shape
    vp = ((vocab + 127) // 128) * 128
    xp = jnp.pad(logits, ((0, 0), (0, vp - vocab)), constant_values=-1e30)
    grid = (n_rows // _ROWS_PER_BLOCK,)
    out = pl.pallas_call(
        _nucleus_block,
        grid=grid,
        in_specs=[pl.BlockSpec((_ROWS_PER_BLOCK, vp), lambda i: (i, 0))],
        out_specs=pl.BlockSpec((_ROWS_PER_BLOCK, vp), lambda i: (i, 0)),
        out_shape=jax.ShapeDtypeStruct((n_rows, vp), jnp.float32),
        scratch_shapes=[pltpu.VMEM((_ROWS_PER_BLOCK, vp), jnp.float32)],
    )(xp)
    return out[:, :vocab]
